# baseline (speedup 1.0000x reference)
import functools

import jax
import jax.numpy as jnp
from jax.experimental import pallas as pl
from jax.experimental.pallas import tpu as pltpu


def _mlp3_kernel(x_ref, w1_ref, b1_ref, w2_ref, b2_ref, w3_ref, b3_ref, o_ref):
    x = x_ref[...].astype(jnp.bfloat16)
    h = jnp.dot(x, w1_ref[...], preferred_element_type=jnp.float32) + b1_ref[...]
    h = jnp.maximum(h, 0.0).astype(jnp.bfloat16)
    h = jnp.dot(h, w2_ref[...], preferred_element_type=jnp.float32) + b2_ref[...]
    h = jnp.maximum(h, 0.0).astype(jnp.bfloat16)
    o_ref[...] = jnp.dot(h, w3_ref[...], preferred_element_type=jnp.float32) + b3_ref[...]


def _fused_mlp(x, params, block_rows):
    w1, b1, w2, b2, w3, b3 = params
    rows, d_in = x.shape
    d_h1 = w1.shape[1]
    d_h2 = w2.shape[1]
    d_out = w3.shape[1]
    w1 = w1.astype(jnp.bfloat16)
    w2 = w2.astype(jnp.bfloat16)
    w3 = w3.astype(jnp.bfloat16)
    b1 = b1.reshape(1, d_h1)
    b2 = b2.reshape(1, d_h2)
    b3 = b3.reshape(1, d_out)

    grid = (rows // block_rows,)
    whole = lambda shape: pl.BlockSpec(shape, lambda i: (0, 0))
    return pl.pallas_call(
        _mlp3_kernel,
        grid=grid,
        in_specs=[
            pl.BlockSpec((block_rows, d_in), lambda i: (i, 0)),
            whole(w1.shape),
            whole(b1.shape),
            whole(w2.shape),
            whole(b2.shape),
            whole(w3.shape),
            whole(b3.shape),
        ],
        out_specs=pl.BlockSpec((block_rows, d_out), lambda i: (i, 0)),
        out_shape=jax.ShapeDtypeStruct((rows, d_out), jnp.float32),
        compiler_params=pltpu.CompilerParams(
            dimension_semantics=("parallel",),
        ),
    )(x, w1, b1, w2, b2, w3, b3)


@jax.jit
def kernel(nodes, edges, global_attr, node_params, edge_params, global_params):
    new_nodes = _fused_mlp(nodes, node_params, block_rows=5000)
    new_edges = _fused_mlp(edges, edge_params, block_rows=16000)
    g = jnp.pad(global_attr, ((0, 7), (0, 0)))
    new_global = _fused_mlp(g, global_params, block_rows=8)[:1]
    return (new_nodes, new_edges, new_global)


# speedup vs baseline: 1.1120x; 1.1120x over previous
import jax
import jax.numpy as jnp
from jax.experimental import pallas as pl
from jax.experimental.pallas import tpu as pltpu

_EDGE_BLOCK = 16000
_NODE_BLOCK = 1000


def _mlp3(x, w1_ref, b1_ref, w2_ref, b2_ref, w3_ref, b3_ref):
    h = jnp.dot(x, w1_ref[...], preferred_element_type=jnp.float32) + b1_ref[...]
    h = jnp.maximum(h, 0.0).astype(jnp.bfloat16)
    h = jnp.dot(h, w2_ref[...], preferred_element_type=jnp.float32) + b2_ref[...]
    h = jnp.maximum(h, 0.0).astype(jnp.bfloat16)
    return jnp.dot(h, w3_ref[...], preferred_element_type=jnp.float32) + b3_ref[...]


def _graph_indep_kernel(
    ex_ref, ew1, eb1, ew2, eb2, ew3, eb3,
    nx_ref, nw1, nb1, nw2, nb2, nw3, nb3,
    gx_ref, gw1, gb1, gw2, gb2, gw3, gb3,
    eo_ref, no_ref, go_ref,
):
    eo_ref[...] = _mlp3(ex_ref[...], ew1, eb1, ew2, eb2, ew3, eb3)
    no_ref[...] = _mlp3(nx_ref[...], nw1, nb1, nw2, nb2, nw3, nb3)

    @pl.when(pl.program_id(0) == 0)
    def _():
        go_ref[...] = _mlp3(gx_ref[...], gw1, gb1, gw2, gb2, gw3, gb3)


def _prep(x, params):
    w1, b1, w2, b2, w3, b3 = params
    return (
        x.astype(jnp.bfloat16),
        w1.astype(jnp.bfloat16), b1.reshape(1, -1),
        w2.astype(jnp.bfloat16), b2.reshape(1, -1),
        w3.astype(jnp.bfloat16), b3.reshape(1, -1),
    )


@jax.jit
def kernel(nodes, edges, global_attr, node_params, edge_params, global_params):
    n_rows = nodes.shape[0]
    e_rows = edges.shape[0]
    d_out = node_params[-1].shape[0]
    grid = (e_rows // _EDGE_BLOCK,)
    assert n_rows // _NODE_BLOCK == grid[0]

    g = jnp.pad(global_attr, ((0, 7), (0, 0)))

    eargs = _prep(edges, edge_params)
    nargs = _prep(nodes, node_params)
    gargs = _prep(g, global_params)

    whole = lambda a: pl.BlockSpec(a.shape, lambda i: (0,) * a.ndim)
    espec = [pl.BlockSpec((_EDGE_BLOCK, edges.shape[1]), lambda i: (i, 0))]
    espec += [whole(a) for a in eargs[1:]]
    nspec = [pl.BlockSpec((_NODE_BLOCK, nodes.shape[1]), lambda i: (i, 0))]
    nspec += [whole(a) for a in nargs[1:]]
    gspec = [whole(a) for a in gargs]

    new_edges, new_nodes, new_global = pl.pallas_call(
        _graph_indep_kernel,
        grid=grid,
        in_specs=espec + nspec + gspec,
        out_specs=[
            pl.BlockSpec((_EDGE_BLOCK, d_out), lambda i: (i, 0)),
            pl.BlockSpec((_NODE_BLOCK, d_out), lambda i: (i, 0)),
            pl.BlockSpec((8, d_out), lambda i: (0, 0)),
        ],
        out_shape=[
            jax.ShapeDtypeStruct((e_rows, d_out), jnp.float32),
            jax.ShapeDtypeStruct((n_rows, d_out), jnp.float32),
            jax.ShapeDtypeStruct((8, d_out), jnp.float32),
        ],
        compiler_params=pltpu.CompilerParams(
            dimension_semantics=("arbitrary",),
        ),
    )(*eargs, *nargs, *gargs)
    return (new_nodes, new_edges, new_global[:1])
